# Initial kernel scaffold; baseline (speedup 1.0000x reference)
import jax
import jax.numpy as jnp
from jax.experimental import pallas as pl


def kernel(indices, table):
    raise NotImplementedError("write your pallas kernel here")


# speedup vs baseline: 7.4125x; 7.4125x over previous
import math

import jax
import jax.numpy as jnp
from jax import lax
from jax.experimental import pallas as pl
from jax.experimental.pallas import tpu as pltpu
from jax.experimental.pallas import tpu_sc as plsc

D_MODEL = 128
V_SIZE = 100000
SCALE = math.sqrt(D_MODEL)

NUM_CORES = 2
NUM_SUBCORES = 16
NUM_WORKERS = NUM_CORES * NUM_SUBCORES

IDX_ROW = 128
ROWS_PER_CHUNK = 4
CHUNK = IDX_ROW * ROWS_PER_CHUNK


def _scale_body(t_ref, o_ref):
    o_ref[...] = t_ref[...] * SCALE


def _scale_table(table):
    blk = 4000
    grid = V_SIZE // blk
    return pl.pallas_call(
        _scale_body,
        grid=(grid,),
        in_specs=[pl.BlockSpec((blk, D_MODEL), lambda i: (i, 0))],
        out_specs=pl.BlockSpec((blk, D_MODEL), lambda i: (i, 0)),
        out_shape=jax.ShapeDtypeStruct((V_SIZE, D_MODEL), jnp.float32),
    )(table)


def _make_sc_gather(num_idx_rows):
    rows_per_worker = num_idx_rows // NUM_WORKERS
    n_chunks = rows_per_worker // ROWS_PER_CHUNK
    out_rows = num_idx_rows * IDX_ROW

    mesh = plsc.VectorSubcoreMesh(core_axis_name="c", subcore_axis_name="s")

    def body(tab_hbm, idx_hbm, out_hbm, idx_v, rows_v, sem):
        wid = lax.axis_index("s") * NUM_CORES + lax.axis_index("c")
        rbase = wid * rows_per_worker

        def chunk(i, carry):
            r = rbase + i * ROWS_PER_CHUNK
            pltpu.sync_copy(idx_hbm.at[pl.ds(r, ROWS_PER_CHUNK)], idx_v)
            copies = [
                pltpu.async_copy(
                    tab_hbm.at[idx_v.at[j]],
                    rows_v.at[pl.ds(j * IDX_ROW, IDX_ROW)],
                    sem,
                )
                for j in range(ROWS_PER_CHUNK)
            ]
            for cp in copies:
                cp.wait()
            pltpu.sync_copy(rows_v, out_hbm.at[pl.ds(r * IDX_ROW, CHUNK)])
            return carry

        lax.fori_loop(0, n_chunks, chunk, 0)

    return pl.kernel(
        body,
        out_type=jax.ShapeDtypeStruct((out_rows, D_MODEL), jnp.float32),
        mesh=mesh,
        scratch_types=[
            pltpu.VMEM((ROWS_PER_CHUNK, IDX_ROW), jnp.int32),
            pltpu.VMEM((CHUNK, D_MODEL), jnp.float32),
            pltpu.SemaphoreType.DMA,
        ],
    )


def kernel(indices, table):
    b0, b1 = indices.shape
    flat = indices.reshape(-1)
    num_idx_rows = flat.shape[0] // IDX_ROW
    idx2d = flat.reshape(num_idx_rows, IDX_ROW)
    scaled = _scale_table(table)
    out = _make_sc_gather(num_idx_rows)(scaled, idx2d)
    return out.reshape(b0, b1, D_MODEL)


# speedup vs baseline: 8.1529x; 1.0999x over previous
import math

import jax
import jax.numpy as jnp
from jax import lax
from jax.experimental import pallas as pl
from jax.experimental.pallas import tpu as pltpu
from jax.experimental.pallas import tpu_sc as plsc

D_MODEL = 128
V_SIZE = 100000
SCALE = math.sqrt(D_MODEL)

NUM_CORES = 2
NUM_SUBCORES = 16
NUM_WORKERS = NUM_CORES * NUM_SUBCORES

IDX_ROW = 128
ROWS_PER_CHUNK = 2
CHUNK = IDX_ROW * ROWS_PER_CHUNK
NBUF = 2


def _scale_body(t_ref, o_ref):
    o_ref[...] = t_ref[...] * SCALE


def _scale_table(table):
    blk = 4000
    grid = V_SIZE // blk
    return pl.pallas_call(
        _scale_body,
        grid=(grid,),
        in_specs=[pl.BlockSpec((blk, D_MODEL), lambda i: (i, 0))],
        out_specs=pl.BlockSpec((blk, D_MODEL), lambda i: (i, 0)),
        out_shape=jax.ShapeDtypeStruct((V_SIZE, D_MODEL), jnp.float32),
    )(table)


def _make_sc_gather(num_idx_rows):
    rows_per_worker = num_idx_rows // NUM_WORKERS
    n_chunks = rows_per_worker // ROWS_PER_CHUNK
    n_groups = n_chunks // NBUF
    out_rows = num_idx_rows * IDX_ROW

    mesh = plsc.VectorSubcoreMesh(core_axis_name="c", subcore_axis_name="s")

    def body(tab_hbm, idx_hbm, out_hbm, idx_all, rows0, rows1, g0, g1, s0, s1):
        wid = lax.axis_index("s") * NUM_CORES + lax.axis_index("c")
        rbase = wid * rows_per_worker
        pltpu.sync_copy(idx_hbm.at[pl.ds(rbase, rows_per_worker)], idx_all)

        rows = [rows0, rows1]
        gsem = [g0, g1]
        ssem = [s0, s1]

        def out_slice(c):
            return out_hbm.at[pl.ds((rbase + c * ROWS_PER_CHUNK) * IDX_ROW, CHUNK)]

        def fire_gather(c, b):
            for j in range(ROWS_PER_CHUNK):
                pltpu.async_copy(
                    tab_hbm.at[idx_all.at[c * ROWS_PER_CHUNK + j]],
                    rows[b].at[pl.ds(j * IDX_ROW, IDX_ROW)],
                    gsem[b],
                )

        def wait_gather(b):
            pltpu.make_async_copy(
                tab_hbm.at[idx_all.at[0]], rows[b].at[pl.ds(0, IDX_ROW)], gsem[b]
            ).wait()
            pltpu.make_async_copy(
                tab_hbm.at[idx_all.at[0]],
                rows[b].at[pl.ds(IDX_ROW, IDX_ROW)],
                gsem[b],
            ).wait()

        def fire_store(c, b):
            pltpu.async_copy(rows[b], out_slice(c), ssem[b])

        def wait_store(c, b):
            pltpu.make_async_copy(rows[b], out_slice(c), ssem[b]).wait()

        for b in range(NBUF):
            fire_gather(b, b)
        for b in range(NBUF):
            wait_gather(b)
            fire_store(b, b)

        def group(g, carry):
            for b in range(NBUF):
                c = g * NBUF + b
                wait_store(c, b)
                fire_gather(c, b)
            for b in range(NBUF):
                c = g * NBUF + b
                wait_gather(b)
                fire_store(c, b)
            return carry

        lax.fori_loop(1, n_groups, group, 0)

        for b in range(NBUF):
            wait_store(0, b)

    return pl.kernel(
        body,
        out_type=jax.ShapeDtypeStruct((out_rows, D_MODEL), jnp.float32),
        mesh=mesh,
        scratch_types=[
            pltpu.VMEM((rows_per_worker, IDX_ROW), jnp.int32),
            pltpu.VMEM((CHUNK, D_MODEL), jnp.float32),
            pltpu.VMEM((CHUNK, D_MODEL), jnp.float32),
            pltpu.SemaphoreType.DMA,
            pltpu.SemaphoreType.DMA,
            pltpu.SemaphoreType.DMA,
            pltpu.SemaphoreType.DMA,
        ],
    )


def kernel(indices, table):
    b0, b1 = indices.shape
    flat = indices.reshape(-1)
    num_idx_rows = flat.shape[0] // IDX_ROW
    idx2d = flat.reshape(num_idx_rows, IDX_ROW)
    scaled = _scale_table(table)
    out = _make_sc_gather(num_idx_rows)(scaled, idx2d)
    return out.reshape(b0, b1, D_MODEL)


# speedup vs baseline: 8.2653x; 1.0138x over previous
import math

import jax
import jax.numpy as jnp
from jax import lax
from jax.experimental import pallas as pl
from jax.experimental.pallas import tpu as pltpu
from jax.experimental.pallas import tpu_sc as plsc

D_MODEL = 128
V_SIZE = 100000
SCALE = math.sqrt(D_MODEL)

NUM_CORES = 2
NUM_SUBCORES = 16
NUM_WORKERS = NUM_CORES * NUM_SUBCORES

IDX_ROW = 128
ROWS_PER_CHUNK = 1
CHUNK = IDX_ROW * ROWS_PER_CHUNK
NBUF = 4


def _scale_body(t_ref, o_ref):
    o_ref[...] = t_ref[...] * SCALE


def _scale_table(table):
    blk = 4000
    grid = V_SIZE // blk
    return pl.pallas_call(
        _scale_body,
        grid=(grid,),
        in_specs=[pl.BlockSpec((blk, D_MODEL), lambda i: (i, 0))],
        out_specs=pl.BlockSpec((blk, D_MODEL), lambda i: (i, 0)),
        out_shape=jax.ShapeDtypeStruct((V_SIZE, D_MODEL), jnp.float32),
    )(table)


def _make_sc_gather(num_idx_rows):
    rows_per_worker = num_idx_rows // NUM_WORKERS
    n_chunks = rows_per_worker // ROWS_PER_CHUNK
    n_groups = n_chunks // NBUF
    out_rows = num_idx_rows * IDX_ROW

    mesh = plsc.VectorSubcoreMesh(core_axis_name="c", subcore_axis_name="s")

    def body(tab_hbm, idx_hbm, out_hbm, idx_all, *bufs):
        wid = lax.axis_index("s") * NUM_CORES + lax.axis_index("c")
        rbase = wid * rows_per_worker
        pltpu.sync_copy(idx_hbm.at[pl.ds(rbase, rows_per_worker)], idx_all)

        rows = list(bufs[:NBUF])
        gsem = list(bufs[NBUF : 2 * NBUF])
        ssem = list(bufs[2 * NBUF :])

        def out_slice(c):
            return out_hbm.at[pl.ds((rbase + c * ROWS_PER_CHUNK) * IDX_ROW, CHUNK)]

        def fire_gather(c, b):
            for j in range(ROWS_PER_CHUNK):
                pltpu.async_copy(
                    tab_hbm.at[idx_all.at[c * ROWS_PER_CHUNK + j]],
                    rows[b].at[pl.ds(j * IDX_ROW, IDX_ROW)],
                    gsem[b],
                )

        def wait_gather(b):
            for j in range(ROWS_PER_CHUNK):
                pltpu.make_async_copy(
                    tab_hbm.at[idx_all.at[0]],
                    rows[b].at[pl.ds(j * IDX_ROW, IDX_ROW)],
                    gsem[b],
                ).wait()

        def fire_store(c, b):
            pltpu.async_copy(rows[b], out_slice(c), ssem[b])

        def wait_store(c, b):
            pltpu.make_async_copy(rows[b], out_slice(c), ssem[b]).wait()

        for b in range(NBUF):
            fire_gather(b, b)
        for b in range(NBUF):
            wait_gather(b)
            fire_store(b, b)

        def group(g, carry):
            for b in range(NBUF):
                c = g * NBUF + b
                wait_store(c, b)
                fire_gather(c, b)
            for b in range(NBUF):
                c = g * NBUF + b
                wait_gather(b)
                fire_store(c, b)
            return carry

        lax.fori_loop(1, n_groups, group, 0)

        for b in range(NBUF):
            wait_store(0, b)

    return pl.kernel(
        body,
        out_type=jax.ShapeDtypeStruct((out_rows, D_MODEL), jnp.float32),
        mesh=mesh,
        scratch_types=(
            [pltpu.VMEM((rows_per_worker, IDX_ROW), jnp.int32)]
            + [pltpu.VMEM((CHUNK, D_MODEL), jnp.float32) for _ in range(NBUF)]
            + [pltpu.SemaphoreType.DMA for _ in range(2 * NBUF)]
        ),
    )


def kernel(indices, table):
    b0, b1 = indices.shape
    flat = indices.reshape(-1)
    num_idx_rows = flat.shape[0] // IDX_ROW
    idx2d = flat.reshape(num_idx_rows, IDX_ROW)
    scaled = _scale_table(table)
    out = _make_sc_gather(num_idx_rows)(scaled, idx2d)
    return out.reshape(b0, b1, D_MODEL)


# speedup vs baseline: 9.1385x; 1.1056x over previous
import functools
import math

import jax
import jax.numpy as jnp
from jax import lax
from jax.experimental import pallas as pl
from jax.experimental.pallas import tpu as pltpu
from jax.experimental.pallas import tpu_sc as plsc

D_MODEL = 128
V_SIZE = 100000
SCALE = math.sqrt(D_MODEL)

NUM_CORES = 2
NUM_SUBCORES = 16
NUM_WORKERS = NUM_CORES * NUM_SUBCORES

IDX_ROW = 128
ROWS_PER_CHUNK = 1
CHUNK = IDX_ROW * ROWS_PER_CHUNK
NBUF = 4


def _make_sc_gather(num_idx_rows):
    rows_per_worker = num_idx_rows // NUM_WORKERS
    n_chunks = rows_per_worker // ROWS_PER_CHUNK
    n_groups = n_chunks // NBUF
    out_rows = num_idx_rows * IDX_ROW

    mesh = plsc.VectorSubcoreMesh(core_axis_name="c", subcore_axis_name="s")

    def body(tab_hbm, idx_hbm, out_hbm, idx_all, *bufs):
        wid = lax.axis_index("s") * NUM_CORES + lax.axis_index("c")
        rbase = wid * rows_per_worker
        pltpu.sync_copy(idx_hbm.at[pl.ds(rbase, rows_per_worker)], idx_all)

        rows = list(bufs[:NBUF])
        gsem = list(bufs[NBUF : 2 * NBUF])
        ssem = list(bufs[2 * NBUF :])

        def out_slice(c):
            return out_hbm.at[pl.ds((rbase + c * ROWS_PER_CHUNK) * IDX_ROW, CHUNK)]

        def fire_gather(c, b):
            for j in range(ROWS_PER_CHUNK):
                pltpu.async_copy(
                    tab_hbm.at[idx_all.at[c * ROWS_PER_CHUNK + j]],
                    rows[b].at[pl.ds(j * IDX_ROW, IDX_ROW)],
                    gsem[b],
                )

        def wait_gather(b):
            for j in range(ROWS_PER_CHUNK):
                pltpu.make_async_copy(
                    tab_hbm.at[idx_all.at[0]],
                    rows[b].at[pl.ds(j * IDX_ROW, IDX_ROW)],
                    gsem[b],
                ).wait()

        def scale_buf(b):
            buf = rows[b]

            @functools.partial(plsc.parallel_loop, 0, CHUNK, unroll=2)
            def _row(i):
                for h in range(D_MODEL // 16):
                    sl = (i, pl.ds(h * 16, 16))
                    buf[sl] = buf[sl] * SCALE

        def fire_store(c, b):
            pltpu.async_copy(rows[b], out_slice(c), ssem[b])

        def wait_store(c, b):
            pltpu.make_async_copy(rows[b], out_slice(c), ssem[b]).wait()

        for b in range(NBUF):
            fire_gather(b, b)
        for b in range(NBUF):
            wait_gather(b)
            scale_buf(b)
            fire_store(b, b)

        def group(g, carry):
            for b in range(NBUF):
                c = g * NBUF + b
                wait_store(c, b)
                fire_gather(c, b)
            for b in range(NBUF):
                c = g * NBUF + b
                wait_gather(b)
                scale_buf(b)
                fire_store(c, b)
            return carry

        lax.fori_loop(1, n_groups, group, 0)

        for b in range(NBUF):
            wait_store(0, b)

    return pl.kernel(
        body,
        out_type=jax.ShapeDtypeStruct((out_rows, D_MODEL), jnp.float32),
        mesh=mesh,
        scratch_types=(
            [pltpu.VMEM((rows_per_worker, IDX_ROW), jnp.int32)]
            + [pltpu.VMEM((CHUNK, D_MODEL), jnp.float32) for _ in range(NBUF)]
            + [pltpu.SemaphoreType.DMA for _ in range(2 * NBUF)]
        ),
    )


def kernel(indices, table):
    b0, b1 = indices.shape
    flat = indices.reshape(-1)
    num_idx_rows = flat.shape[0] // IDX_ROW
    idx2d = flat.reshape(num_idx_rows, IDX_ROW)
    out = _make_sc_gather(num_idx_rows)(table, idx2d)
    return out.reshape(b0, b1, D_MODEL)
